# baseline (speedup 1.0000x reference)
import functools

import jax
import jax.numpy as jnp
from jax import lax
from jax.experimental import pallas as pl
from jax.experimental.pallas import tpu as pltpu
from jax.experimental.pallas import tpu_sc as plsc

BATCH = 16384
EMBED_DIM = 64
NUM_ENT = 1000000
SPLIT = 499712
SLICE = 62464
TAIL_PAD = 640
REL_PAD = 1024
BUF0_SIZE = SPLIT + REL_PAD
BUF1_SIZE = SPLIT + TAIL_PAD
NUM_REL = 1000
NUM_CORES = 2
NUM_SUBCORES = 16
ROWS_PER_TILE = BATCH // NUM_SUBCORES
DIMS_PER_CORE = EMBED_DIM // NUM_CORES
LANES = 16
IDX_CHUNK = 128
NUM_IDX_CHUNKS = ROWS_PER_TILE // IDX_CHUNK
GROUPS_PER_CHUNK = IDX_CHUNK // LANES


def _sc_partials_body(hid_hbm, tid_hbm, hA_hbm, hB_hbm, tA_hbm, tB_hbm,
                      rx_hbm, entT_hbm, tailT_hbm, relT_hbm, out_hbm,
                      hid_v, tid_v, hA_v, hB_v, tA_v, tB_v, rx_v,
                      hbufA, hbufB, tbufA, tbufB, rbuf, scores_v,
                      spcol0, spcol1, sem_s, sem_g):
    cid = lax.axis_index("c")
    sid = lax.axis_index("s")

    def stage0_refs(j, t):
        if t < 8:
            sl = pl.ds(t * SLICE, SLICE)
            return entT_hbm.at[j, sl], spcol0.at[sl]
        return relT_hbm.at[j], spcol0.at[pl.ds(SPLIT, REL_PAD)]

    def stage1_refs(j, t):
        if t < 8:
            return (entT_hbm.at[j, pl.ds(SPLIT + t * SLICE, SLICE)],
                    spcol1.at[pl.ds(t * SLICE, SLICE)])
        return tailT_hbm.at[j], spcol1.at[pl.ds(SPLIT, TAIL_PAD)]

    def issue(refs_fn, j):
        for t in range(9):
            @pl.when(sid == t)
            def _(t=t):
                src, dst = refs_fn(j, t)
                pltpu.async_copy(src, dst, sem_s)

    def drain(refs_fn, j):
        for t in range(9):
            @pl.when(sid == t)
            def _(t=t):
                src, dst = refs_fn(j, t)
                pltpu.make_async_copy(src, dst, sem_s).wait()

    for src, dst in ((hid_hbm, hid_v), (tid_hbm, tid_v), (hA_hbm, hA_v),
                     (hB_hbm, hB_v), (tA_hbm, tA_v), (tB_hbm, tB_v),
                     (rx_hbm, rx_v)):
        pltpu.sync_copy(src.at[sid], dst)

    def zinit(g, carry):
        scores_v[pl.ds(g * LANES, LANES)] = jnp.zeros((LANES,), jnp.float32)
        return carry
    lax.fori_loop(0, ROWS_PER_TILE // LANES, zinit, 0)

    j0 = cid * DIMS_PER_CORE
    issue(stage0_refs, j0)

    def dim(d, carry):
        j = j0 + d

        drain(stage0_refs, j)
        plsc.subcore_barrier()
        issue(stage1_refs, j)

        copies = []
        for k in range(NUM_IDX_CHUNKS):
            copies.append(pltpu.async_copy(spcol0.at[hA_v.at[k]],
                                           hbufA.at[k], sem_g))
            copies.append(pltpu.async_copy(spcol0.at[tA_v.at[k]],
                                           tbufA.at[k], sem_g))
            copies.append(pltpu.async_copy(spcol0.at[rx_v.at[k]],
                                           rbuf.at[k], sem_g))
        for c in copies:
            c.wait()

        drain(stage1_refs, j)
        plsc.subcore_barrier()

        @pl.when(d < DIMS_PER_CORE - 1)
        def _prefetch():
            issue(stage0_refs, j + 1)

        copies = []
        for k in range(NUM_IDX_CHUNKS):
            copies.append(pltpu.async_copy(spcol1.at[hB_v.at[k]],
                                           hbufB.at[k], sem_g))
            copies.append(pltpu.async_copy(spcol1.at[tB_v.at[k]],
                                           tbufB.at[k], sem_g))
        for c in copies:
            c.wait()

        for k in range(NUM_IDX_CHUNKS):
            for g in range(GROUPS_PER_CHUNK):
                sl = pl.ds(g * LANES, LANES)
                hv = jnp.where(hid_v[k, sl] < SPLIT, hbufA[k, sl],
                               hbufB[k, sl])
                tv = jnp.where(tid_v[k, sl] < SPLIT, tbufA[k, sl],
                               tbufB[k, sl])
                rv = rbuf[k, sl]
                row0 = k * IDX_CHUNK + g * LANES
                scores_v[pl.ds(row0, LANES)] = (
                    scores_v[pl.ds(row0, LANES)] + hv * rv * tv)
        return carry

    lax.fori_loop(0, DIMS_PER_CORE, dim, 0)

    pltpu.sync_copy(scores_v, out_hbm.at[cid, sid])


_sc_partials = functools.partial(
    pl.kernel,
    mesh=plsc.VectorSubcoreMesh(core_axis_name="c", subcore_axis_name="s"),
    out_type=jax.ShapeDtypeStruct((NUM_CORES, NUM_SUBCORES, ROWS_PER_TILE),
                                  jnp.float32),
    scratch_types=(
        [pltpu.VMEM((NUM_IDX_CHUNKS, IDX_CHUNK), jnp.int32)] * 7 +
        [pltpu.VMEM((NUM_IDX_CHUNKS, IDX_CHUNK), jnp.float32)] * 5 +
        [pltpu.VMEM((ROWS_PER_TILE,), jnp.float32),
         pltpu.VMEM_SHARED((BUF0_SIZE,), jnp.float32),
         pltpu.VMEM_SHARED((BUF1_SIZE,), jnp.float32),
         pltpu.SemaphoreType.DMA,
         pltpu.SemaphoreType.DMA]
    ),
    compiler_params=pltpu.CompilerParams(needs_layout_passes=False),
)(_sc_partials_body)


def _softmax_body(x_ref, o_ref):
    scores = x_ref[0] + x_ref[1]
    m = jnp.max(scores)
    e = jnp.exp(scores - m)
    o_ref[...] = e * (1.0 / jnp.sum(e))


_softmax = pl.pallas_call(
    _softmax_body,
    out_shape=jax.ShapeDtypeStruct((128, 128), jnp.float32),
)


def _tiles(x):
    return x.reshape(NUM_SUBCORES, NUM_IDX_CHUNKS, IDX_CHUNK)


def kernel(head_ids, rel_ids, tail_ids, entity_embeddings, relation_embeddings):
    hid = head_ids.astype(jnp.int32)
    rid = rel_ids.astype(jnp.int32)
    tid = tail_ids.astype(jnp.int32)
    entT = entity_embeddings.T
    tailT = jnp.pad(entT[:, 2 * SPLIT:],
                    ((0, 0), (0, TAIL_PAD - (NUM_ENT - 2 * SPLIT))))
    relT = jnp.pad(relation_embeddings.T, ((0, 0), (0, REL_PAD - NUM_REL)))
    partials = _sc_partials(
        _tiles(hid), _tiles(tid),
        _tiles(jnp.minimum(hid, SPLIT - 1)),
        _tiles(jnp.maximum(hid - SPLIT, 0)),
        _tiles(jnp.minimum(tid, SPLIT - 1)),
        _tiles(jnp.maximum(tid - SPLIT, 0)),
        _tiles(rid + SPLIT),
        entT, tailT, relT)
    return _softmax(partials.reshape(2, 128, 128)).reshape(BATCH)


# speedup vs baseline: 2.8626x; 2.8626x over previous
import functools

import jax
import jax.numpy as jnp
from jax import lax
from jax.experimental import pallas as pl
from jax.experimental.pallas import tpu as pltpu
from jax.experimental.pallas import tpu_sc as plsc

BATCH = 16384
EMBED_DIM = 64
NUM_ENT = 1000000
MAIN_ENT = 999424
SLICE = 124928
TAIL_PAD = 640
REL_PAD = 1024
REL_OFF = MAIN_ENT + TAIL_PAD
SPCOL_SIZE = REL_OFF + REL_PAD
NUM_REL = 1000
NUM_CORES = 2
NUM_SUBCORES = 16
ROWS_PER_TILE = BATCH // NUM_SUBCORES
DIMS_PER_CORE = EMBED_DIM // NUM_CORES
LANES = 16
IDX_CHUNK = 128
NUM_IDX_CHUNKS = ROWS_PER_TILE // IDX_CHUNK
GROUPS_PER_CHUNK = IDX_CHUNK // LANES


def _sc_partials_body(hid_hbm, tid_hbm, rx_hbm, entT_hbm, tailT_hbm,
                      relT_hbm, out_hbm, hid_v, tid_v, rx_v,
                      hbuf, tbuf, rbuf, scores_v, spcol, sem_s, sem_g):
    cid = lax.axis_index("c")
    sid = lax.axis_index("s")

    def stage_refs(j, t):
        if t < 8:
            sl = pl.ds(t * SLICE, SLICE)
            return entT_hbm.at[j, sl], spcol.at[sl]
        if t == 8:
            return tailT_hbm.at[j], spcol.at[pl.ds(MAIN_ENT, TAIL_PAD)]
        return relT_hbm.at[j], spcol.at[pl.ds(REL_OFF, REL_PAD)]

    def issue(j):
        for t in range(10):
            @pl.when(sid == t)
            def _(t=t):
                src, dst = stage_refs(j, t)
                pltpu.async_copy(src, dst, sem_s)

    def drain(j):
        for t in range(10):
            @pl.when(sid == t)
            def _(t=t):
                src, dst = stage_refs(j, t)
                pltpu.make_async_copy(src, dst, sem_s).wait()

    pltpu.sync_copy(hid_hbm.at[sid], hid_v)
    pltpu.sync_copy(tid_hbm.at[sid], tid_v)
    pltpu.sync_copy(rx_hbm.at[sid], rx_v)

    def zinit(g, carry):
        scores_v[pl.ds(g * LANES, LANES)] = jnp.zeros((LANES,), jnp.float32)
        return carry
    lax.fori_loop(0, ROWS_PER_TILE // LANES, zinit, 0)

    j0 = cid * DIMS_PER_CORE
    issue(j0)

    def dim(d, carry):
        j = j0 + d

        drain(j)
        plsc.subcore_barrier()

        copies = []
        for k in range(NUM_IDX_CHUNKS):
            copies.append(pltpu.async_copy(spcol.at[hid_v.at[k]],
                                           hbuf.at[k], sem_g))
            copies.append(pltpu.async_copy(spcol.at[tid_v.at[k]],
                                           tbuf.at[k], sem_g))
            copies.append(pltpu.async_copy(spcol.at[rx_v.at[k]],
                                           rbuf.at[k], sem_g))
        for c in copies:
            c.wait()

        plsc.subcore_barrier()

        @pl.when(d < DIMS_PER_CORE - 1)
        def _prefetch():
            issue(j + 1)

        for k in range(NUM_IDX_CHUNKS):
            for g in range(GROUPS_PER_CHUNK):
                sl = pl.ds(g * LANES, LANES)
                row0 = k * IDX_CHUNK + g * LANES
                scores_v[pl.ds(row0, LANES)] = (
                    scores_v[pl.ds(row0, LANES)]
                    + hbuf[k, sl] * rbuf[k, sl] * tbuf[k, sl])
        return carry

    lax.fori_loop(0, DIMS_PER_CORE, dim, 0)

    pltpu.sync_copy(scores_v, out_hbm.at[cid, sid])


_sc_partials = functools.partial(
    pl.kernel,
    mesh=plsc.VectorSubcoreMesh(core_axis_name="c", subcore_axis_name="s"),
    out_type=jax.ShapeDtypeStruct((NUM_CORES, NUM_SUBCORES, ROWS_PER_TILE),
                                  jnp.float32),
    scratch_types=[
        pltpu.VMEM((NUM_IDX_CHUNKS, IDX_CHUNK), jnp.int32),
        pltpu.VMEM((NUM_IDX_CHUNKS, IDX_CHUNK), jnp.int32),
        pltpu.VMEM((NUM_IDX_CHUNKS, IDX_CHUNK), jnp.int32),
        pltpu.VMEM((NUM_IDX_CHUNKS, IDX_CHUNK), jnp.float32),
        pltpu.VMEM((NUM_IDX_CHUNKS, IDX_CHUNK), jnp.float32),
        pltpu.VMEM((NUM_IDX_CHUNKS, IDX_CHUNK), jnp.float32),
        pltpu.VMEM((ROWS_PER_TILE,), jnp.float32),
        pltpu.VMEM_SHARED((SPCOL_SIZE,), jnp.float32),
        pltpu.SemaphoreType.DMA,
        pltpu.SemaphoreType.DMA,
    ],
    compiler_params=pltpu.CompilerParams(needs_layout_passes=False),
)(_sc_partials_body)


def _softmax_body(x_ref, o_ref):
    scores = x_ref[0] + x_ref[1]
    m = jnp.max(scores)
    e = jnp.exp(scores - m)
    o_ref[...] = e * (1.0 / jnp.sum(e))


_softmax = pl.pallas_call(
    _softmax_body,
    out_shape=jax.ShapeDtypeStruct((128, 128), jnp.float32),
)


def _tiles(x):
    return x.reshape(NUM_SUBCORES, NUM_IDX_CHUNKS, IDX_CHUNK)


def kernel(head_ids, rel_ids, tail_ids, entity_embeddings, relation_embeddings):
    hid = head_ids.astype(jnp.int32)
    rid = rel_ids.astype(jnp.int32)
    tid = tail_ids.astype(jnp.int32)
    entT = entity_embeddings.T
    tailT = jnp.pad(entT[:, MAIN_ENT:],
                    ((0, 0), (0, TAIL_PAD - (NUM_ENT - MAIN_ENT))))
    relT = jnp.pad(relation_embeddings.T, ((0, 0), (0, REL_PAD - NUM_REL)))
    partials = _sc_partials(_tiles(hid), _tiles(tid), _tiles(rid + REL_OFF),
                            entT, tailT, relT)
    return _softmax(partials.reshape(2, 128, 128)).reshape(BATCH)


# speedup vs baseline: 2.8633x; 1.0003x over previous
import functools

import jax
import jax.numpy as jnp
from jax import lax
from jax.experimental import pallas as pl
from jax.experimental.pallas import tpu as pltpu
from jax.experimental.pallas import tpu_sc as plsc

BATCH = 16384
EMBED_DIM = 64
NUM_ENT = 1000000
MAIN_ENT = 999424
SLICE = 249856
TAIL_PAD = 640
REL_PAD = 1024
REL_OFF = MAIN_ENT + TAIL_PAD
SPCOL_SIZE = REL_OFF + REL_PAD
NUM_REL = 1000
NUM_CORES = 2
NUM_SUBCORES = 16
ROWS_PER_TILE = BATCH // NUM_SUBCORES
DIMS_PER_CORE = EMBED_DIM // NUM_CORES
LANES = 16
IDX_CHUNK = 128
NUM_IDX_CHUNKS = ROWS_PER_TILE // IDX_CHUNK
GROUPS_PER_CHUNK = IDX_CHUNK // LANES


def _sc_partials_body(hid_hbm, tid_hbm, rx_hbm, entT_hbm, tailT_hbm,
                      relT_hbm, out_hbm, hid_v, tid_v, rx_v,
                      hbuf, tbuf, rbuf, scores_v, spcol, sem_s, sem_g):
    cid = lax.axis_index("c")
    sid = lax.axis_index("s")

    def stage_refs(j, t):
        if t < 4:
            sl = pl.ds(t * SLICE, SLICE)
            return entT_hbm.at[j, sl], spcol.at[sl]
        if t == 4:
            return tailT_hbm.at[j], spcol.at[pl.ds(MAIN_ENT, TAIL_PAD)]
        return relT_hbm.at[j], spcol.at[pl.ds(REL_OFF, REL_PAD)]

    def issue(j):
        for t in range(6):
            @pl.when(sid == t)
            def _(t=t):
                src, dst = stage_refs(j, t)
                pltpu.async_copy(src, dst, sem_s)

    def drain(j):
        for t in range(6):
            @pl.when(sid == t)
            def _(t=t):
                src, dst = stage_refs(j, t)
                pltpu.make_async_copy(src, dst, sem_s).wait()

    pltpu.sync_copy(hid_hbm.at[sid], hid_v)
    pltpu.sync_copy(tid_hbm.at[sid], tid_v)
    pltpu.sync_copy(rx_hbm.at[sid], rx_v)

    def zinit(g, carry):
        scores_v[pl.ds(g * LANES, LANES)] = jnp.zeros((LANES,), jnp.float32)
        return carry
    lax.fori_loop(0, ROWS_PER_TILE // LANES, zinit, 0)

    j0 = cid * DIMS_PER_CORE
    issue(j0)

    def dim(d, carry):
        j = j0 + d

        drain(j)
        plsc.subcore_barrier()

        copies = []
        for k in range(NUM_IDX_CHUNKS):
            copies.append(pltpu.async_copy(spcol.at[hid_v.at[k]],
                                           hbuf.at[k], sem_g))
            copies.append(pltpu.async_copy(spcol.at[tid_v.at[k]],
                                           tbuf.at[k], sem_g))
            copies.append(pltpu.async_copy(spcol.at[rx_v.at[k]],
                                           rbuf.at[k], sem_g))
        for c in copies:
            c.wait()

        plsc.subcore_barrier()

        @pl.when(d < DIMS_PER_CORE - 1)
        def _prefetch():
            issue(j + 1)

        for k in range(NUM_IDX_CHUNKS):
            for g in range(GROUPS_PER_CHUNK):
                sl = pl.ds(g * LANES, LANES)
                row0 = k * IDX_CHUNK + g * LANES
                scores_v[pl.ds(row0, LANES)] = (
                    scores_v[pl.ds(row0, LANES)]
                    + hbuf[k, sl] * rbuf[k, sl] * tbuf[k, sl])
        return carry

    lax.fori_loop(0, DIMS_PER_CORE, dim, 0)

    pltpu.sync_copy(scores_v, out_hbm.at[cid, sid])


_sc_partials = functools.partial(
    pl.kernel,
    mesh=plsc.VectorSubcoreMesh(core_axis_name="c", subcore_axis_name="s"),
    out_type=jax.ShapeDtypeStruct((NUM_CORES, NUM_SUBCORES, ROWS_PER_TILE),
                                  jnp.float32),
    scratch_types=[
        pltpu.VMEM((NUM_IDX_CHUNKS, IDX_CHUNK), jnp.int32),
        pltpu.VMEM((NUM_IDX_CHUNKS, IDX_CHUNK), jnp.int32),
        pltpu.VMEM((NUM_IDX_CHUNKS, IDX_CHUNK), jnp.int32),
        pltpu.VMEM((NUM_IDX_CHUNKS, IDX_CHUNK), jnp.float32),
        pltpu.VMEM((NUM_IDX_CHUNKS, IDX_CHUNK), jnp.float32),
        pltpu.VMEM((NUM_IDX_CHUNKS, IDX_CHUNK), jnp.float32),
        pltpu.VMEM((ROWS_PER_TILE,), jnp.float32),
        pltpu.VMEM_SHARED((SPCOL_SIZE,), jnp.float32),
        pltpu.SemaphoreType.DMA,
        pltpu.SemaphoreType.DMA,
    ],
    compiler_params=pltpu.CompilerParams(needs_layout_passes=False),
)(_sc_partials_body)


def _softmax_body(x_ref, o_ref):
    scores = x_ref[0] + x_ref[1]
    m = jnp.max(scores)
    e = jnp.exp(scores - m)
    o_ref[...] = e * (1.0 / jnp.sum(e))


_softmax = pl.pallas_call(
    _softmax_body,
    out_shape=jax.ShapeDtypeStruct((128, 128), jnp.float32),
)


def _tiles(x):
    return x.reshape(NUM_SUBCORES, NUM_IDX_CHUNKS, IDX_CHUNK)


def kernel(head_ids, rel_ids, tail_ids, entity_embeddings, relation_embeddings):
    hid = head_ids.astype(jnp.int32)
    rid = rel_ids.astype(jnp.int32)
    tid = tail_ids.astype(jnp.int32)
    entT = entity_embeddings.T
    tailT = jnp.pad(entT[:, MAIN_ENT:],
                    ((0, 0), (0, TAIL_PAD - (NUM_ENT - MAIN_ENT))))
    relT = jnp.pad(relation_embeddings.T, ((0, 0), (0, REL_PAD - NUM_REL)))
    partials = _sc_partials(_tiles(hid), _tiles(tid), _tiles(rid + REL_OFF),
                            entT, tailT, relT)
    return _softmax(partials.reshape(2, 128, 128)).reshape(BATCH)
